# baseline (speedup 1.0000x reference)
import functools

import jax
import jax.numpy as jnp
from jax import lax
from jax.experimental import pallas as pl
from jax.experimental.pallas import tpu as pltpu
from jax.experimental.pallas import tpu_sc as plsc

_U = 25000
_I = 25000
_N = _U + _I
_E = 1600000
_D = 32
_LAYERS = 3

_NW = 32
_EW = _E // _NW
_B = 128
_S = 8
_NSC = 49
_NB = _S * _NSC
_EWP = _NB * _B
_NP = 50176
_G = 49
_AR = _G * 32
_NCH = _EWP // _B + 1
_STG = 288
_BIGC = 16
_BIG = _BIGC * _B


def _i16(x):
    return jnp.full((16,), x, jnp.int32)


def _bin_body(cols, rows, vals, bcol, bval, bloc, counts,
              colv, rowv, valv, stc, stv, stl, fbc, fbv, fbl, cntout, semf):
    cid = lax.axis_index("c")
    sid = lax.axis_index("s")
    w = sid * 2 + cid
    iota = lax.iota(jnp.int32, 16)
    zv = jnp.zeros((16,), jnp.int32)

    def _drain(op, hp):
        @pl.when(op >= 0)
        def _():
            pltpu.make_async_copy(fbc.at[op], bcol.at[w, op, hp], semf).wait()
            pltpu.make_async_copy(fbv.at[op], bval.at[w, op, hp], semf).wait()
            pltpu.make_async_copy(fbl.at[op], bloc.at[w, op, hp], semf).wait()

    def _gbody(jj, g, cr):
        c0, c1 = cr
        col16 = colv[jj, pl.ds(g * 16, 16)]
        row16 = rowv[jj, pl.ds(g * 16, 16)]
        val16 = valv[jj, pl.ds(g * 16, 16)]
        own = jnp.bitwise_and(jnp.right_shift(row16, 5), 31)
        loc = jnp.bitwise_or(
            jnp.left_shift(jnp.right_shift(row16, 10), 5),
            jnp.bitwise_and(row16, 31))
        for o in range(32):
            mask = own == o
            ch = c0 if o < 16 else c1
            cnt_o = ch[o % 16] + (o * _STG)
            plsc.store_compressed(stc.at[pl.ds(cnt_o, 16)], col16,
                                  mask=mask)
            plsc.store_compressed(stv.at[pl.ds(cnt_o, 16)], val16,
                                  mask=mask)
            plsc.store_compressed(stl.at[pl.ds(cnt_o, 16)], loc,
                                  mask=mask)
            pop = plsc.all_reduce_population_count(mask)
            upd = jnp.where(iota == (o % 16), pop, zv)
            if o < 16:
                c0 = c0 + upd
            else:
                c1 = c1 + upd
        return (c0, c1)

    def _flush(o, cr):
        c0, c1, h0, h1, op, hp = cr
        lane = jnp.bitwise_and(o, 15)
        hi = jnp.right_shift(o, 4)
        lmask = iota == lane
        cvec = jnp.where(hi == 0, c0, c1)
        cnt_o = jnp.sum(jnp.where(lmask, cvec, zv))
        hvec = jnp.where(hi == 0, h0, h1)
        h_o = jnp.sum(jnp.where(lmask, hvec, zv))
        pred = cnt_o >= _B

        @pl.when(pred)
        def _():
            _drain(op, hp)
            base = o * _STG
            for q in range(8):
                s = pl.ds(q * 16, 16)
                sf = pl.ds(base + q * 16, 16)
                fbc[o, s] = stc[sf]
                fbv[o, s] = stv[sf]
                fbl[o, s] = stl[sf]
            pltpu.async_copy(fbc.at[o], bcol.at[w, o, h_o], semf)
            pltpu.async_copy(fbv.at[o], bval.at[w, o, h_o], semf)
            pltpu.async_copy(fbl.at[o], bloc.at[w, o, h_o], semf)
            for q in range(8):
                s = pl.ds(base + q * 16, 16)
                s2 = pl.ds(base + _B + q * 16, 16)
                stc[s] = stc[s2]
                stv[s] = stv[s2]
                stl[s] = stl[s2]

        dec = jnp.where(pred, 1, 0)
        decv = jnp.where(lmask, dec, 0)
        hi0 = hi == 0
        c0 = jnp.where(hi0, c0 - decv * _B, c0)
        c1 = jnp.where(hi0, c1, c1 - decv * _B)
        h0 = jnp.where(hi0, h0 + decv, h0)
        h1 = jnp.where(hi0, h1, h1 + decv)
        op = jnp.where(pred, o, op)
        hp = jnp.where(pred, h_o, hp)
        return (c0, c1, h0, h1, op, hp)

    def _bbody(jj, cr):
        c0, c1, h0, h1, op, hp = cr
        c0, c1 = lax.fori_loop(
            0, _B // 16, functools.partial(_gbody, jj), (c0, c1))
        return lax.fori_loop(0, 32, _flush, (c0, c1, h0, h1, op, hp))

    def _cbody(j, cr):
        pltpu.sync_copy(cols.at[w, j], colv)
        pltpu.sync_copy(rows.at[w, j], rowv)
        pltpu.sync_copy(vals.at[w, j], valv)
        return lax.fori_loop(0, _S, _bbody, cr)

    init = (zv, zv, zv, zv, jnp.int32(-1), jnp.int32(0))
    c0, c1, h0, h1, op, hp = lax.fori_loop(0, _NSC, _cbody, init)
    _drain(op, hp)

    def _fin(o, cr):
        lane = jnp.bitwise_and(o, 15)
        hvec = jnp.where(jnp.right_shift(o, 4) == 0, h0, h1)
        h_o = jnp.sum(jnp.where(iota == lane, hvec, zv))
        pltpu.sync_copy(stc.at[pl.ds(o * _STG, _B)], bcol.at[w, o, h_o])
        pltpu.sync_copy(stv.at[pl.ds(o * _STG, _B)], bval.at[w, o, h_o])
        pltpu.sync_copy(stl.at[pl.ds(o * _STG, _B)], bloc.at[w, o, h_o])
        return cr

    lax.fori_loop(0, 32, _fin, 0)
    cntout[pl.ds(0, 16)] = h0 * _B + c0
    cntout[pl.ds(16, 16)] = h1 * _B + c1
    pltpu.sync_copy(cntout, counts.at[w])


def _make_bin_kernel():
    mesh = plsc.VectorSubcoreMesh(core_axis_name="c", subcore_axis_name="s")
    return functools.partial(
        pl.kernel,
        mesh=mesh,
        compiler_params=pltpu.CompilerParams(use_tc_tiling_on_sc=False,
                                             needs_layout_passes=False),
        out_type=(
            jax.ShapeDtypeStruct((_NW, _NW, _NCH, _B), jnp.int32),
            jax.ShapeDtypeStruct((_NW, _NW, _NCH, _B), jnp.float32),
            jax.ShapeDtypeStruct((_NW, _NW, _NCH, _B), jnp.int32),
            jax.ShapeDtypeStruct((_NW, _NW), jnp.int32),
        ),
        scratch_types=[
            pltpu.VMEM((_S, _B), jnp.int32),
            pltpu.VMEM((_S, _B), jnp.int32),
            pltpu.VMEM((_S, _B), jnp.float32),
            pltpu.VMEM((32 * _STG,), jnp.int32),
            pltpu.VMEM((32 * _STG,), jnp.float32),
            pltpu.VMEM((32 * _STG,), jnp.int32),
            pltpu.VMEM((32, _B), jnp.int32),
            pltpu.VMEM((32, _B), jnp.float32),
            pltpu.VMEM((32, _B), jnp.int32),
            pltpu.VMEM((32,), jnp.int32),
            pltpu.SemaphoreType.DMA,
        ],
    )(_bin_body)


def _layer_body(ego, bcol, bval, bloc, cntT, out, acc, cntv,
                cb0, vb0, lb0, cb1, vb1, lb1, m0, m1, m2, m3,
                co, vo, lo_, mo, sb0, sb1, sg0, sg1, sg2, sg3, so):
    cid = lax.axis_index("c")
    sid = lax.axis_index("s")
    t = sid * 2 + cid
    iota = lax.iota(jnp.int32, 16)
    zv = jnp.zeros((16,), jnp.int32)
    zf = jnp.zeros((16,), jnp.float32)

    def _zb(i, c):
        acc[i, pl.ds(0, 16)] = zf
        acc[i, pl.ds(16, 16)] = zf
        return c

    lax.fori_loop(0, _AR, _zb, 0)
    pltpu.sync_copy(cntT.at[t], cntv)
    cv0 = cntv[pl.ds(0, 16)]
    cv1 = cntv[pl.ds(16, 16)]

    sets = ((cb0, vb0, lb0, sb0), (cb1, vb1, lb1, sb1))
    msgs = (m0, m1, m2, m3)
    gsems = (sg0, sg1, sg2, sg3)

    def _issue_big(p, s):
        cb, vb, lb, sb = sets[s]
        pltpu.async_copy(bcol.at[p, t, pl.ds(0, _BIGC)], cb, sb)
        pltpu.async_copy(bval.at[p, t, pl.ds(0, _BIGC)], vb, sb)
        pltpu.async_copy(bloc.at[p, t, pl.ds(0, _BIGC)], lb, sb)

    def _process(p, s):
        cb, vb, lb, sb = sets[s]
        lane = jnp.bitwise_and(p, 15)
        cvec = jnp.where(jnp.right_shift(p, 4) == 0, cv0, cv1)
        c_p = jnp.sum(jnp.where(iota == lane, cvec, zv))
        c_fast = jnp.minimum(c_p, _BIG)
        nch = jnp.right_shift(c_fast + 127, 7)

        pltpu.make_async_copy(bcol.at[p, t, pl.ds(0, _BIGC)], cb, sb).wait()
        pltpu.make_async_copy(bval.at[p, t, pl.ds(0, _BIGC)], vb, sb).wait()
        pltpu.make_async_copy(bloc.at[p, t, pl.ds(0, _BIGC)], lb, sb).wait()

        def _san(k):
            def _sg(g, c):
                sl = pl.ds(g * 16, 16)
                m = (k * _B + g * 16 + iota) < c_fast
                cb[k, sl] = jnp.clip(cb[k, sl], 0, _NP - 1)
                lb[k, sl] = jnp.clip(lb[k, sl], 0, _AR - 1)
                vb[k, sl] = jnp.where(m, vb[k, sl], 0.0)
                return c

            lax.fori_loop(0, _B // 16, _sg, 0)

        def _compute(k, m):
            def _cg(g, c):
                sl = pl.ds(g * 16, 16)
                l16 = lb[k, sl]
                v16 = vb[k, sl]
                for e2 in range(16):
                    e = g * 16 + e2
                    l = l16[e2]
                    v = v16[e2]
                    plsc.addupdate(acc.at[l, pl.ds(0, 16)],
                                   m[e, pl.ds(0, 16)] * v)
                    plsc.addupdate(acc.at[l, pl.ds(16, 16)],
                                   m[e, pl.ds(16, 16)] * v)
                return c

            lax.fori_loop(0, _B // 16, _cg, 0)

        for j in range(4):
            kj = jnp.int32(j)

            @pl.when(kj < nch)
            def _(j=j, kj=kj):
                _san(kj)
                pltpu.async_copy(ego.at[cb.at[kj]], msgs[j], gsems[j])

        def _kbody(kk, c):
            for j in range(4):
                k = kk * 4 + j

                @pl.when(k < nch)
                def _(j=j, k=k):
                    pltpu.make_async_copy(ego.at[cb.at[k]], msgs[j],
                                          gsems[j]).wait()
                    _compute(k, msgs[j])
                    kn = k + 4

                    @pl.when(kn < nch)
                    def _(j=j, kn=kn):
                        _san(kn)
                        pltpu.async_copy(ego.at[cb.at[kn]], msgs[j],
                                         gsems[j])
            return c

        lax.fori_loop(0, _BIGC // 4, _kbody, 0)

        nslow = jnp.right_shift(c_p - c_fast + 127, 7)

        def _sbody(k2, c):
            kc = _BIGC + k2
            pltpu.sync_copy(bcol.at[p, t, kc], co)
            pltpu.sync_copy(bval.at[p, t, kc], vo)
            pltpu.sync_copy(bloc.at[p, t, kc], lo_)

            def _sg2(g, cc):
                sl = pl.ds(g * 16, 16)
                m = (_BIG + k2 * _B + g * 16 + iota) < c_p
                co[sl] = jnp.clip(co[sl], 0, _NP - 1)
                lo_[sl] = jnp.clip(lo_[sl], 0, _AR - 1)
                vo[sl] = jnp.where(m, vo[sl], 0.0)
                return cc

            lax.fori_loop(0, _B // 16, _sg2, 0)
            pltpu.async_copy(ego.at[co], mo, so).wait()

            def _cg2(g, cc):
                sl = pl.ds(g * 16, 16)
                l16 = lo_[sl]
                v16 = vo[sl]
                for e2 in range(16):
                    e = g * 16 + e2
                    l = l16[e2]
                    v = v16[e2]
                    plsc.addupdate(acc.at[l, pl.ds(0, 16)],
                                   mo[e, pl.ds(0, 16)] * v)
                    plsc.addupdate(acc.at[l, pl.ds(16, 16)],
                                   mo[e, pl.ds(16, 16)] * v)
                return cc

            lax.fori_loop(0, _B // 16, _cg2, 0)
            return c

        lax.fori_loop(0, nslow, _sbody, 0)

    _issue_big(0, 0)

    def _pbody(pp, c):
        p0 = pp * 2
        _issue_big(p0 + 1, 1)
        _process(p0, 0)

        @pl.when(pp < (_NW // 2 - 1))
        def _():
            _issue_big(p0 + 2, 0)

        _process(p0 + 1, 1)
        return c

    lax.fori_loop(0, _NW // 2, _pbody, 0)

    def _dr(g, c):
        pltpu.sync_copy(acc.at[pl.ds(g * 32, 32)],
                        out.at[pl.ds(g * 1024 + t * 32, 32)])
        return c

    lax.fori_loop(0, _G, _dr, 0)


def _make_layer_kernel():
    mesh = plsc.VectorSubcoreMesh(core_axis_name="c", subcore_axis_name="s")
    return functools.partial(
        pl.kernel,
        mesh=mesh,
        compiler_params=pltpu.CompilerParams(use_tc_tiling_on_sc=False,
                                             needs_layout_passes=False),
        out_type=jax.ShapeDtypeStruct((_NP, _D), jnp.float32),
        scratch_types=[
            pltpu.VMEM((_AR, _D), jnp.float32),
            pltpu.VMEM((32,), jnp.int32),
            pltpu.VMEM((_BIGC, _B), jnp.int32),
            pltpu.VMEM((_BIGC, _B), jnp.float32),
            pltpu.VMEM((_BIGC, _B), jnp.int32),
            pltpu.VMEM((_BIGC, _B), jnp.int32),
            pltpu.VMEM((_BIGC, _B), jnp.float32),
            pltpu.VMEM((_BIGC, _B), jnp.int32),
            pltpu.VMEM((_B, _D), jnp.float32),
            pltpu.VMEM((_B, _D), jnp.float32),
            pltpu.VMEM((_B, _D), jnp.float32),
            pltpu.VMEM((_B, _D), jnp.float32),
            pltpu.VMEM((_B,), jnp.int32),
            pltpu.VMEM((_B,), jnp.float32),
            pltpu.VMEM((_B,), jnp.int32),
            pltpu.VMEM((_B, _D), jnp.float32),
            pltpu.SemaphoreType.DMA,
            pltpu.SemaphoreType.DMA,
            pltpu.SemaphoreType.DMA,
            pltpu.SemaphoreType.DMA,
            pltpu.SemaphoreType.DMA,
            pltpu.SemaphoreType.DMA,
            pltpu.SemaphoreType.DMA,
        ],
    )(_layer_body)


def _mean_body(e0, e1, e2, e3, o_ref):
    o_ref[...] = (e0[...] + e1[...] + e2[...] + e3[...]) * 0.25


def _mean4(e0, e1, e2, e3):
    rb = 6272
    spec = pl.BlockSpec((rb, _D), lambda i: (i, 0))
    return pl.pallas_call(
        _mean_body,
        grid=(_NP // rb,),
        in_specs=[spec] * 4,
        out_specs=spec,
        out_shape=jax.ShapeDtypeStruct((_NP, _D), jnp.float32),
    )(e0, e1, e2, e3)


def kernel(user_emb, item_emb, edge_vals, edge_index):
    ego0 = jnp.concatenate([user_emb, item_emb], axis=0)
    ego0 = jnp.pad(ego0, ((0, _NP - _N), (0, 0)))

    pad = _EWP - _EW
    cols = edge_index[1].reshape(_NW, _EW)
    rows = edge_index[0].reshape(_NW, _EW)
    vals = edge_vals.reshape(_NW, _EW)
    zi = jnp.zeros((_NW, pad), jnp.int32)
    zf = jnp.zeros((_NW, pad), jnp.float32)
    pd = jnp.broadcast_to((jnp.arange(pad, dtype=jnp.int32) * 283) % _N,
                          (_NW, pad))
    cols = jnp.concatenate([cols, zi], axis=1).reshape(_NW, _NSC, _S, _B)
    rows = jnp.concatenate([rows, pd], axis=1).reshape(_NW, _NSC, _S, _B)
    vals = jnp.concatenate([vals, zf], axis=1).reshape(_NW, _NSC, _S, _B)

    bcol, bval, bloc, counts = _make_bin_kernel()(cols, rows, vals)
    cntT = counts.T

    layer = _make_layer_kernel()
    egos = [ego0]
    e = ego0
    for _ in range(_LAYERS):
        e = layer(e, bcol, bval, bloc, cntT)
        egos.append(e)

    all_e = _mean4(egos[0], egos[1], egos[2], egos[3])
    return (all_e[:_U], all_e[_U:_N])
